# baseline (speedup 1.0000x reference)
import dataclasses
import functools

import jax
import jax.numpy as jnp
import numpy as np
from jax import lax
from jax.experimental import pallas as pl
from jax.experimental.pallas import tpu as pltpu
from jax.experimental.pallas import tpu_sc as plsc

N = 10000
E = 320000
NFEAT = 128
NHID = 64
NCLASS = 40
NLAYERS = 8
ALPHA = 0.1
LAMDA = 0.5

NC = 2
NS = 16
NW = NC * NS
L = 16
EPW = E // NW
CHUNK = 80
NCHUNK = EPW // CHUNK
BLK = 200
NBLK = N // BLK
NBUF = 5


def _spmm_sc(h, src3, dst3, w2):
    mesh = plsc.VectorSubcoreMesh(core_axis_name="c", subcore_axis_name="s")
    cp = pltpu.CompilerParams()
    for f, v in (("needs_layout_passes", False),
                 ("use_tc_tiling_on_sc", False)):
        if f in pltpu.CompilerParams.__dataclass_fields__:
            cp = dataclasses.replace(cp, **{f: v})

    @functools.partial(
        pl.kernel,
        mesh=mesh,
        compiler_params=cp,
        out_type=jax.ShapeDtypeStruct((NC, N, NHID), jnp.float32),
        scratch_types=[
            pltpu.VMEM((NCHUNK, CHUNK), jnp.int32),
            pltpu.VMEM((NCHUNK, CHUNK), jnp.int32),
            pltpu.VMEM((EPW,), jnp.float32),
            pltpu.VMEM((NBUF, CHUNK, NHID), jnp.bfloat16),
            pltpu.VMEM((NBUF, CHUNK, NHID), jnp.float32),
            pltpu.VMEM_SHARED((N, NHID), jnp.float32),
            pltpu.SemaphoreType.DMA((NBUF,)),
            pltpu.SemaphoreType.DMA((NBUF,)),
            pltpu.SemaphoreType.DMA,
            pltpu.SemaphoreType.DMA,
        ],
    )
    def k(h_hbm, src_hbm, dst_hbm, w_hbm, out_hbm, src_v, dst_v, w_v,
          rows_g, rows_s, agg_sh, sem_g, sem_s, sem_p, sem_z):
        cid = lax.axis_index("c")
        sid = lax.axis_index("s")
        wid = cid * NS + sid
        nzb = N // CHUNK

        pltpu.async_copy(src_hbm.at[wid], src_v, sem_p)
        pltpu.async_copy(dst_hbm.at[wid], dst_v, sem_p)
        pltpu.async_copy(w_hbm.at[wid], w_v, sem_p)

        @pl.loop(0, CHUNK)
        def _(r):
            for c in range(NHID // L):
                rows_s[0, r, pl.ds(c * L, L)] = jnp.zeros((L,), jnp.float32)

        for j in range((nzb + NS - 1) // NS):
            b = j * NS + sid

            @pl.when(b < nzb)
            def _():
                pltpu.async_copy(rows_s.at[0],
                                 agg_sh.at[pl.ds(b * CHUNK, CHUNK)], sem_z)

        pltpu.make_async_copy(src_hbm.at[wid], src_v, sem_p).wait()
        pltpu.make_async_copy(dst_hbm.at[wid], dst_v, sem_p).wait()
        pltpu.make_async_copy(w_hbm.at[wid], w_v, sem_p).wait()
        for b in range(NBUF):
            pltpu.async_copy(h_hbm.at[src_v.at[b]], rows_g.at[b], sem_g.at[b])

        for j in range((nzb + NS - 1) // NS):
            b = j * NS + sid

            @pl.when(b < nzb)
            def _():
                pltpu.make_async_copy(
                    rows_s.at[0], agg_sh.at[pl.ds(b * CHUNK, CHUNK)],
                    sem_z).wait()

        plsc.subcore_barrier()

        @pl.loop(0, NCHUNK, step=NBUF)
        def _(k0):
            for b in range(NBUF):
                ck = k0 + b
                pltpu.make_async_copy(h_hbm.at[src_v.at[ck]], rows_g.at[b],
                                      sem_g.at[b]).wait()

                @pl.when(k0 > 0)
                def _():
                    pltpu.make_async_copy(
                        rows_s.at[b], agg_sh.at[dst_v.at[ck - NBUF]],
                        sem_s.at[b]).wait()

                @plsc.parallel_loop(0, CHUNK, unroll=16)
                def _(r):
                    idx = jnp.broadcast_to(ck * CHUNK + r, (L,)).astype(
                        jnp.int32)
                    wv = plsc.load_gather(w_v, [idx])
                    for c in range(NHID // (2 * L)):
                        x = rows_g[b, r, pl.ds(c * 2 * L, 2 * L)]
                        lo, hi = plsc.unpack(
                            x, format=plsc.PackFormat.INTERLEAVED)
                        rows_s[b, r, pl.ds(c * 2 * L, L)] = lo * wv
                        rows_s[b, r, pl.ds(c * 2 * L + L, L)] = hi * wv

                pltpu.async_copy(rows_s.at[b], agg_sh.at[dst_v.at[ck]],
                                 sem_s.at[b], add=True)

                @pl.when(ck + NBUF < NCHUNK)
                def _():
                    pltpu.async_copy(h_hbm.at[src_v.at[ck + NBUF]],
                                     rows_g.at[b], sem_g.at[b])

        for b in range(NBUF):
            pltpu.make_async_copy(rows_s.at[b],
                                  agg_sh.at[dst_v.at[NCHUNK - NBUF + b]],
                                  sem_s.at[b]).wait()

        plsc.subcore_barrier()

        for j in range((NBLK + NS - 1) // NS):
            b = j * NS + sid

            @pl.when(b < NBLK)
            def _():
                pltpu.async_copy(agg_sh.at[pl.ds(b * BLK, BLK)],
                                 out_hbm.at[cid, pl.ds(b * BLK, BLK)], sem_z)

        for j in range((NBLK + NS - 1) // NS):
            b = j * NS + sid

            @pl.when(b < NBLK)
            def _():
                pltpu.make_async_copy(
                    agg_sh.at[pl.ds(b * BLK, BLK)],
                    out_hbm.at[cid, pl.ds(b * BLK, BLK)], sem_z).wait()

    return k(h, src3, dst3, w2)


PERM = np.empty((NHID,), dtype=np.int32)
for _g in range(2):
    for _m in range(16):
        for _q in range(2):
            PERM[_g * 32 + 2 * _m + _q] = _g * 32 + _q * 16 + _m


def _fc1(x, W1, b1):
    def body(x_ref, w_ref, b_ref, wp_ref, bp_ref, o_ref, ob_ref):
        o_ref[...] = jax.nn.relu(
            jnp.dot(x_ref[...], w_ref[...], preferred_element_type=jnp.float32)
            + b_ref[...])
        ob_ref[...] = jax.nn.relu(
            jnp.dot(x_ref[...], wp_ref[...],
                    preferred_element_type=jnp.float32)
            + bp_ref[...]).astype(jnp.bfloat16)

    return pl.pallas_call(
        body,
        out_shape=(jax.ShapeDtypeStruct((N, NHID), jnp.float32),
                   jax.ShapeDtypeStruct((N, NHID), jnp.bfloat16)),
    )(x, W1, b1.reshape(1, NHID), W1[:, PERM], b1[PERM].reshape(1, NHID))


def _layer_update(p, h0, Mp):
    def body(p_ref, h0_ref, m_ref, ob_ref):
        s = (1.0 - ALPHA) * (p_ref[0] + p_ref[1]) + ALPHA * h0_ref[...]
        ob_ref[...] = jax.nn.relu(
            jnp.dot(s, m_ref[...],
                    preferred_element_type=jnp.float32)).astype(jnp.bfloat16)

    return pl.pallas_call(
        body,
        out_shape=jax.ShapeDtypeStruct((N, NHID), jnp.bfloat16),
    )(p, h0, Mp)


def _layer_update_out(p, h0, M, W2, b2):
    def body(p_ref, h0_ref, m_ref, w_ref, b_ref, o_ref):
        s = (1.0 - ALPHA) * (p_ref[0] + p_ref[1]) + ALPHA * h0_ref[...]
        h = jax.nn.relu(jnp.dot(s, m_ref[...],
                                preferred_element_type=jnp.float32))
        logits = (jnp.dot(h, w_ref[...],
                          preferred_element_type=jnp.float32) + b_ref[...])
        m = jnp.max(logits, axis=1, keepdims=True)
        lse = jnp.log(jnp.sum(jnp.exp(logits - m), axis=1, keepdims=True)) + m
        o_ref[...] = logits - lse

    return pl.pallas_call(
        body,
        out_shape=jax.ShapeDtypeStruct((N, NCLASS), jnp.float32),
    )(p, h0, M, W2, b2.reshape(1, NCLASS))


def kernel(x, edge_index, edge_weight, W1, b1, Wc, W2, b2):
    src3 = edge_index[0].reshape(NW, NCHUNK, CHUNK)
    dst3 = edge_index[1].reshape(NW, NCHUNK, CHUNK)
    w2 = edge_weight.reshape(NW, EPW)
    thetas = np.log(LAMDA / (np.arange(1, NLAYERS + 1)) + 1.0).astype(np.float32)
    eye = jnp.eye(NHID, dtype=jnp.float32)
    M = (jnp.asarray(thetas)[:, None, None] * Wc
         + (1.0 - jnp.asarray(thetas))[:, None, None] * eye[None])

    Mp = M[:, :, PERM]

    h0, hb = _fc1(x, W1, b1)
    for i in range(NLAYERS - 1):
        p = _spmm_sc(hb, src3, dst3, w2)
        hb = _layer_update(p, h0, Mp[i])
    p = _spmm_sc(hb, src3, dst3, w2)
    return _layer_update_out(p, h0, M[NLAYERS - 1], W2, b2)


# speedup vs baseline: 1.0179x; 1.0179x over previous
import dataclasses
import functools

import jax
import jax.numpy as jnp
import numpy as np
from jax import lax
from jax.experimental import pallas as pl
from jax.experimental.pallas import tpu as pltpu
from jax.experimental.pallas import tpu_sc as plsc

N = 10000
E = 320000
NFEAT = 128
NHID = 64
NCLASS = 40
NLAYERS = 8
ALPHA = 0.1
LAMDA = 0.5

NC = 2
NS = 16
NW = NC * NS
L = 16
EPW = E // NW
CHUNK = 80
NCHUNK = EPW // CHUNK
BLK = 200
NBLK = N // BLK
NBUF = 5


def _spmm_sc(h, src3, dst3, w2):
    mesh = plsc.VectorSubcoreMesh(core_axis_name="c", subcore_axis_name="s")
    cp = pltpu.CompilerParams()
    for f, v in (("needs_layout_passes", False),
                 ("use_tc_tiling_on_sc", False)):
        if f in pltpu.CompilerParams.__dataclass_fields__:
            cp = dataclasses.replace(cp, **{f: v})

    @functools.partial(
        pl.kernel,
        mesh=mesh,
        compiler_params=cp,
        out_type=jax.ShapeDtypeStruct((NC, N, NHID), jnp.float32),
        scratch_types=[
            pltpu.VMEM((NCHUNK, CHUNK), jnp.int32),
            pltpu.VMEM((NCHUNK, CHUNK), jnp.int32),
            pltpu.VMEM((EPW,), jnp.float32),
            pltpu.VMEM((NBUF, CHUNK, NHID), jnp.bfloat16),
            pltpu.VMEM((NBUF, CHUNK, NHID), jnp.float32),
            pltpu.VMEM_SHARED((N, NHID), jnp.float32),
            pltpu.SemaphoreType.DMA((NBUF,)),
            pltpu.SemaphoreType.DMA((NBUF,)),
            pltpu.SemaphoreType.DMA,
            pltpu.SemaphoreType.DMA,
        ],
    )
    def k(h_hbm, src_hbm, dst_hbm, w_hbm, out_hbm, src_v, dst_v, w_v,
          rows_g, rows_s, agg_sh, sem_g, sem_s, sem_p, sem_z):
        cid = lax.axis_index("c")
        sid = lax.axis_index("s")
        wid = cid * NS + sid
        nzb = N // CHUNK

        pltpu.async_copy(src_hbm.at[wid], src_v, sem_p)
        pltpu.async_copy(dst_hbm.at[wid], dst_v, sem_p)
        pltpu.async_copy(w_hbm.at[wid], w_v, sem_p)

        @pl.loop(0, CHUNK)
        def _(r):
            for c in range(NHID // L):
                rows_s[0, r, pl.ds(c * L, L)] = jnp.zeros((L,), jnp.float32)

        for j in range((nzb + NS - 1) // NS):
            b = j * NS + sid

            @pl.when(b < nzb)
            def _():
                pltpu.async_copy(rows_s.at[0],
                                 agg_sh.at[pl.ds(b * CHUNK, CHUNK)], sem_z)

        pltpu.make_async_copy(src_hbm.at[wid], src_v, sem_p).wait()
        pltpu.make_async_copy(dst_hbm.at[wid], dst_v, sem_p).wait()
        pltpu.make_async_copy(w_hbm.at[wid], w_v, sem_p).wait()
        for b in range(NBUF):
            pltpu.async_copy(h_hbm.at[src_v.at[b]], rows_g.at[b], sem_g.at[b])

        for j in range((nzb + NS - 1) // NS):
            b = j * NS + sid

            @pl.when(b < nzb)
            def _():
                pltpu.make_async_copy(
                    rows_s.at[0], agg_sh.at[pl.ds(b * CHUNK, CHUNK)],
                    sem_z).wait()

        plsc.subcore_barrier()

        @pl.loop(0, NCHUNK, step=NBUF)
        def _(k0):
            for b in range(NBUF):
                ck = k0 + b
                pltpu.make_async_copy(h_hbm.at[src_v.at[ck]], rows_g.at[b],
                                      sem_g.at[b]).wait()

                @pl.when(k0 > 0)
                def _():
                    pltpu.make_async_copy(
                        rows_s.at[b], agg_sh.at[dst_v.at[ck - NBUF]],
                        sem_s.at[b]).wait()

                @plsc.parallel_loop(0, CHUNK, unroll=8)
                def _(r):
                    idx = jnp.broadcast_to(ck * CHUNK + r, (L,)).astype(
                        jnp.int32)
                    wv = plsc.load_gather(w_v, [idx])
                    for c in range(NHID // (2 * L)):
                        x = rows_g[b, r, pl.ds(c * 2 * L, 2 * L)]
                        lo, hi = plsc.unpack(
                            x, format=plsc.PackFormat.INTERLEAVED)
                        rows_s[b, r, pl.ds(c * 2 * L, L)] = lo * wv
                        rows_s[b, r, pl.ds(c * 2 * L + L, L)] = hi * wv

                pltpu.async_copy(rows_s.at[b], agg_sh.at[dst_v.at[ck]],
                                 sem_s.at[b], add=True)

                @pl.when(ck + NBUF < NCHUNK)
                def _():
                    pltpu.async_copy(h_hbm.at[src_v.at[ck + NBUF]],
                                     rows_g.at[b], sem_g.at[b])

        for b in range(NBUF):
            pltpu.make_async_copy(rows_s.at[b],
                                  agg_sh.at[dst_v.at[NCHUNK - NBUF + b]],
                                  sem_s.at[b]).wait()

        plsc.subcore_barrier()

        for j in range((NBLK + NS - 1) // NS):
            b = j * NS + sid

            @pl.when(b < NBLK)
            def _():
                pltpu.async_copy(agg_sh.at[pl.ds(b * BLK, BLK)],
                                 out_hbm.at[cid, pl.ds(b * BLK, BLK)], sem_z)

        for j in range((NBLK + NS - 1) // NS):
            b = j * NS + sid

            @pl.when(b < NBLK)
            def _():
                pltpu.make_async_copy(
                    agg_sh.at[pl.ds(b * BLK, BLK)],
                    out_hbm.at[cid, pl.ds(b * BLK, BLK)], sem_z).wait()

    return k(h, src3, dst3, w2)


PERM = np.empty((NHID,), dtype=np.int32)
for _g in range(2):
    for _m in range(16):
        for _q in range(2):
            PERM[_g * 32 + 2 * _m + _q] = _g * 32 + _q * 16 + _m


def _fc1(x, W1, b1):
    def body(x_ref, w_ref, b_ref, wp_ref, bp_ref, o_ref, ob_ref):
        o_ref[...] = jax.nn.relu(
            jnp.dot(x_ref[...], w_ref[...], preferred_element_type=jnp.float32)
            + b_ref[...])
        ob_ref[...] = jax.nn.relu(
            jnp.dot(x_ref[...], wp_ref[...],
                    preferred_element_type=jnp.float32)
            + bp_ref[...]).astype(jnp.bfloat16)

    return pl.pallas_call(
        body,
        out_shape=(jax.ShapeDtypeStruct((N, NHID), jnp.float32),
                   jax.ShapeDtypeStruct((N, NHID), jnp.bfloat16)),
    )(x, W1, b1.reshape(1, NHID), W1[:, PERM], b1[PERM].reshape(1, NHID))


def _layer_update(p, h0, Mp):
    def body(p_ref, h0_ref, m_ref, ob_ref):
        s = (1.0 - ALPHA) * (p_ref[0] + p_ref[1]) + ALPHA * h0_ref[...]
        ob_ref[...] = jax.nn.relu(
            jnp.dot(s, m_ref[...],
                    preferred_element_type=jnp.float32)).astype(jnp.bfloat16)

    return pl.pallas_call(
        body,
        out_shape=jax.ShapeDtypeStruct((N, NHID), jnp.bfloat16),
    )(p, h0, Mp)


def _layer_update_out(p, h0, M, W2, b2):
    def body(p_ref, h0_ref, m_ref, w_ref, b_ref, o_ref):
        s = (1.0 - ALPHA) * (p_ref[0] + p_ref[1]) + ALPHA * h0_ref[...]
        h = jax.nn.relu(jnp.dot(s, m_ref[...],
                                preferred_element_type=jnp.float32))
        logits = (jnp.dot(h, w_ref[...],
                          preferred_element_type=jnp.float32) + b_ref[...])
        m = jnp.max(logits, axis=1, keepdims=True)
        lse = jnp.log(jnp.sum(jnp.exp(logits - m), axis=1, keepdims=True)) + m
        o_ref[...] = logits - lse

    return pl.pallas_call(
        body,
        out_shape=jax.ShapeDtypeStruct((N, NCLASS), jnp.float32),
    )(p, h0, M, W2, b2.reshape(1, NCLASS))


def kernel(x, edge_index, edge_weight, W1, b1, Wc, W2, b2):
    src3 = edge_index[0].reshape(NW, NCHUNK, CHUNK)
    dst3 = edge_index[1].reshape(NW, NCHUNK, CHUNK)
    w2 = edge_weight.reshape(NW, EPW)
    thetas = np.log(LAMDA / (np.arange(1, NLAYERS + 1)) + 1.0).astype(np.float32)
    eye = jnp.eye(NHID, dtype=jnp.float32)
    M = (jnp.asarray(thetas)[:, None, None] * Wc
         + (1.0 - jnp.asarray(thetas))[:, None, None] * eye[None])

    Mp = M[:, :, PERM]

    h0, hb = _fc1(x, W1, b1)
    for i in range(NLAYERS - 1):
        p = _spmm_sc(hb, src3, dst3, w2)
        hb = _layer_update(p, h0, Mp[i])
    p = _spmm_sc(hb, src3, dst3, w2)
    return _layer_update_out(p, h0, M[NLAYERS - 1], W2, b2)
